# baseline (speedup 1.0000x reference)
import jax
import jax.numpy as jnp
from jax.experimental import pallas as pl


def _router_body(x_ref, w_ref, b_ref, o_ref):
    logits = jax.lax.dot_general(
        x_ref[...], w_ref[...],
        dimension_numbers=(((1,), (1,)), ((), ())),
        preferred_element_type=jnp.float32,
    ) + b_ref[...]
    m = jnp.max(logits, axis=-1, keepdims=True)
    e = jnp.exp(logits - m)
    o_ref[...] = e / jnp.sum(e, axis=-1, keepdims=True)


def kernel(x, W, b):
    tokens, hidden = x.shape
    experts = W.shape[0]
    bt = 256
    grid = (tokens // bt,)
    b2 = b.reshape(1, experts)
    return pl.pallas_call(
        _router_body,
        grid=grid,
        in_specs=[
            pl.BlockSpec((bt, hidden), lambda i: (i, 0)),
            pl.BlockSpec((experts, hidden), lambda i: (0, 0)),
            pl.BlockSpec((1, experts), lambda i: (0, 0)),
        ],
        out_specs=pl.BlockSpec((bt, experts), lambda i: (i, 0)),
        out_shape=jax.ShapeDtypeStruct((tokens, experts), jnp.float32),
    )(x, W, b2)


# speedup vs baseline: 1.2059x; 1.2059x over previous
import jax
import jax.numpy as jnp
from jax.experimental import pallas as pl


def _router_body(x_ref, w_ref, b_ref, o_ref):
    logits = jax.lax.dot_general(
        x_ref[...], w_ref[...],
        dimension_numbers=(((1,), (1,)), ((), ())),
        preferred_element_type=jnp.float32,
    ) + b_ref[...]
    m = jnp.max(logits, axis=-1, keepdims=True)
    e = jnp.exp(logits - m)
    o_ref[...] = e / jnp.sum(e, axis=-1, keepdims=True)


def kernel(x, W, b):
    tokens, hidden = x.shape
    experts = W.shape[0]
    bt = 512
    grid = (tokens // bt,)
    b2 = b.reshape(1, experts)
    return pl.pallas_call(
        _router_body,
        grid=grid,
        in_specs=[
            pl.BlockSpec((bt, hidden), lambda i: (i, 0)),
            pl.BlockSpec((experts, hidden), lambda i: (0, 0)),
            pl.BlockSpec((1, experts), lambda i: (0, 0)),
        ],
        out_specs=pl.BlockSpec((bt, experts), lambda i: (i, 0)),
        out_shape=jax.ShapeDtypeStruct((tokens, experts), jnp.float32),
    )(x, W, b2)
